# baseline (speedup 1.0000x reference)
import numpy as np
import jax
import jax.numpy as jnp
from jax import lax
from jax.experimental import pallas as pl

_V = 8192
_K = 1024
_NC = 8192
_INTERP = False

_F = jnp.float32
_I = jnp.int32


def _build_cand_indices():
    ci = np.zeros(_NC, np.int32)
    cj = np.zeros(_NC, np.int32)
    valid = np.zeros(_NC, np.int32)
    t = 0
    for a in range(_K):
        nb = _K // (a + 1)
        if nb == 0:
            break
        ci[t:t + nb] = a
        cj[t:t + nb] = np.arange(nb, dtype=np.int32)
        valid[t:t + nb] = 1
        t += nb
    assert _K <= t <= _NC, t
    return ci, cj, valid


_CI_NP, _CJ_NP, _CVALID_NP = _build_cand_indices()


def _t128(row_chunk):
    return lax.dot_general(row_chunk, jnp.ones((1, 1), _F),
                           (((0,), (0,)), ((), ())),
                           precision=lax.Precision.HIGHEST,
                           preferred_element_type=_F)


def _topk_rank_scan(xr_ref, kr_get, ki_col_fn, n):
    nch = n // 128
    rk_iota = lax.broadcasted_iota(_I, (1, _K), 1)

    def c_step(c, accs):
        va, ka = accs
        xi = _t128(xr_ref[:, pl.ds(c * 128, 128)])
        ki = ki_col_fn(c)

        def d_step(d, acc):
            xj = xr_ref[:, pl.ds(d * 128, 128)]
            kj = kr_get(d)
            beats = (xj > xi) | ((xj == xi) & (kj < ki))
            return acc + beats.astype(_I)

        acc = lax.fori_loop(0, nch, d_step, jnp.zeros((128, 128), _I))
        rank = jnp.sum(acc, axis=1, keepdims=True)
        hit = rank == rk_iota
        va = va + jnp.sum(jnp.where(hit, xi, 0.0), axis=0, keepdims=True)
        ka = ka + jnp.sum(jnp.where(hit, ki, 0.0), axis=0, keepdims=True)
        return va, ka

    return lax.fori_loop(0, nch, c_step,
                         (jnp.zeros((1, _K), _F), jnp.zeros((1, _K), _F)))


def _topk_iota_body(xr_ref, vals_ref, idx_ref):
    def kr_get(d):
        return (lax.broadcasted_iota(_I, (1, 128), 1) + d * 128).astype(_F)

    def ki_col(c):
        return (lax.broadcasted_iota(_I, (128, 1), 0) + c * 128).astype(_F)

    va, ka = _topk_rank_scan(xr_ref, kr_get, ki_col, _V)
    vals_ref[...] = va
    idx_ref[...] = ka.astype(_I)


def _topk_keyed_body(xr_ref, kr_ref, vals_ref, key_ref):
    def kr_get(d):
        return kr_ref[:, pl.ds(d * 128, 128)]

    def ki_col(c):
        return _t128(kr_ref[:, pl.ds(c * 128, 128)])

    va, ka = _topk_rank_scan(xr_ref, kr_get, ki_col, _NC)
    vals_ref[...] = va
    key_ref[...] = ka.astype(_I)


def _cand_body(vr_ref, ur_ref, jr_ref, cir_ref, cjr_ref, valr_ref,
               cv_ref, ck_ref):
    ci = cir_ref[...]
    cj = cjr_ref[...]
    val = valr_ref[...]

    def step(kb, acc):
        av, au, aj = acc
        kc = lax.broadcasted_iota(_I, (128, 1), 0) + kb * 128
        vcb = _t128(vr_ref[:, pl.ds(kb * 128, 128)])
        ucb = _t128(ur_ref[:, pl.ds(kb * 128, 128)])
        jcb = _t128(jr_ref[:, pl.ds(kb * 128, 128)])
        mi = kc == ci
        mj = kc == cj
        av = av + jnp.sum(jnp.where(mi, vcb, 0.0), axis=0, keepdims=True)
        au = au + jnp.sum(jnp.where(mj, ucb, 0.0), axis=0, keepdims=True)
        aj = aj + jnp.sum(jnp.where(mj, jcb, 0.0), axis=0, keepdims=True)
        return av, au, aj

    av, au, aj = lax.fori_loop(
        0, _K // 128, step,
        (jnp.zeros((1, _NC), _F), jnp.zeros((1, _NC), _F),
         jnp.zeros((1, _NC), _F)))
    ok = val != 0
    cv_ref[...] = jnp.where(ok, av * au, -1.0)
    pad = (lax.broadcasted_iota(_I, (1, _NC), 1) + _V * _K).astype(_F)
    ck_ref[...] = jnp.where(ok, ci.astype(_F) * _V + aj, pad)


def _resolve_body(i1r_ref, sk2r_ref, sk3r_ref, s0_ref, s1_ref, s2_ref):
    sk3 = sk3r_ref[...]
    r3 = jnp.floor(sk3 * (1.0 / _V))
    c3 = sk3 - r3 * _V

    def step(kb, acc):
        e, s1 = acc
        kc = (lax.broadcasted_iota(_I, (128, 1), 0) + kb * 128).astype(_F)
        sk2c = _t128(sk2r_ref[:, pl.ds(kb * 128, 128)])
        a2c = jnp.floor(sk2c * (1.0 / _V))
        b2c = sk2c - a2c * _V
        m = kc == r3
        e = e + jnp.sum(jnp.where(m, a2c, 0.0), axis=0, keepdims=True)
        s1 = s1 + jnp.sum(jnp.where(m, b2c, 0.0), axis=0, keepdims=True)
        return e, s1

    e, s1 = lax.fori_loop(0, _K // 128, step,
                          (jnp.zeros((1, _K), _F), jnp.zeros((1, _K), _F)))

    def step2(kb, s0):
        kc = (lax.broadcasted_iota(_I, (128, 1), 0) + kb * 128).astype(_F)
        i1c = _t128(i1r_ref[:, pl.ds(kb * 128, 128)])
        m2 = kc == e
        return s0 + jnp.sum(jnp.where(m2, i1c, 0.0), axis=0, keepdims=True)

    s0 = lax.fori_loop(0, _K // 128, step2, jnp.zeros((1, _K), _F))
    s0_ref[...] = s0.astype(_I)
    s1_ref[...] = s1.astype(_I)
    s2_ref[...] = c3.astype(_I)


_RPB = 8


def _mat_body(v3_ref, s0_ref, s1_ref, s2_ref, p3_ref, probs_ref, syms_ref):
    p3v = p3_ref[...]
    t = lax.broadcasted_iota(_I, (1, 4 * _V), 1)
    cpat = t & 3
    jpat = t >> 2
    for rr in range(_RPB):
        v3 = v3_ref[0, 0, rr]
        probs_ref[pl.ds(rr * _V, _V)] = (v3 * p3v).reshape(_V)
        s0 = s0_ref[0, 0, rr]
        s1 = s1_ref[0, 0, rr]
        s2 = s2_ref[0, 0, rr]
        out = jnp.where(cpat == 0, s0,
                        jnp.where(cpat == 1, s1,
                                  jnp.where(cpat == 2, s2, jpat)))
        syms_ref[pl.ds(rr * 4 * _V, 4 * _V)] = out.reshape(4 * _V)


def kernel(p0, p1, p2, p3, k):
    del k

    topk_iota = pl.pallas_call(
        _topk_iota_body,
        out_shape=[jax.ShapeDtypeStruct((1, _K), _F),
                   jax.ShapeDtypeStruct((1, _K), _I)],
        interpret=_INTERP,
    )
    topk_keyed = pl.pallas_call(
        _topk_keyed_body,
        out_shape=[jax.ShapeDtypeStruct((1, _K), _F),
                   jax.ShapeDtypeStruct((1, _K), _I)],
        interpret=_INTERP,
    )
    cand = pl.pallas_call(
        _cand_body,
        out_shape=[jax.ShapeDtypeStruct((1, _NC), _F),
                   jax.ShapeDtypeStruct((1, _NC), _F)],
        interpret=_INTERP,
    )
    resolve = pl.pallas_call(
        _resolve_body,
        out_shape=[jax.ShapeDtypeStruct((1, _K), _I)] * 3,
        interpret=_INTERP,
    )
    mat = pl.pallas_call(
        _mat_body,
        grid=(_K // _RPB,),
        in_specs=[
            pl.BlockSpec((1, 1, _RPB), lambda i: (i, 0, 0)),
            pl.BlockSpec((1, 1, _RPB), lambda i: (i, 0, 0)),
            pl.BlockSpec((1, 1, _RPB), lambda i: (i, 0, 0)),
            pl.BlockSpec((1, 1, _RPB), lambda i: (i, 0, 0)),
            pl.BlockSpec((1, _V), lambda i: (0, 0)),
        ],
        out_specs=[
            pl.BlockSpec((_V * _RPB,), lambda i: (i,)),
            pl.BlockSpec((4 * _V * _RPB,), lambda i: (i,)),
        ],
        out_shape=[jax.ShapeDtypeStruct((_K * _V,), _F),
                   jax.ShapeDtypeStruct((_K * _V * 4,), _I)],
        interpret=_INTERP,
    )

    cir = jnp.asarray(_CI_NP).reshape(1, _NC)
    cjr = jnp.asarray(_CJ_NP).reshape(1, _NC)
    valr = jnp.asarray(_CVALID_NP).reshape(1, _NC)

    def row(a):
        return a.reshape(1, a.size)

    v1, i1 = topk_iota(row(p0))
    u1, j1 = topk_iota(row(p1))
    u2, j2 = topk_iota(row(p2))

    cv2, ck2 = cand(v1, u1, j1.astype(_F), cir, cjr, valr)
    v2, sk2 = topk_keyed(cv2, ck2)

    cv3, ck3 = cand(v2, u2, j2.astype(_F), cir, cjr, valr)
    v3, sk3 = topk_keyed(cv3, ck3)

    s0, s1, s2 = resolve(i1.astype(_F), sk2.astype(_F), sk3.astype(_F))

    def grp(a):
        return a.reshape(_K // _RPB, 1, _RPB)

    probs1d, syms1d = mat(grp(v3), grp(s0), grp(s1), grp(s2),
                          p3.reshape(1, _V))
    return probs1d, syms1d.reshape(_K * _V, 4)


# speedup vs baseline: 3.6567x; 3.6567x over previous
import numpy as np
import jax
import jax.numpy as jnp
from jax import lax
from jax.experimental import pallas as pl

_V = 8192
_K = 1024
_NC = 8192
_INTERP = False

_F = jnp.float32
_I = jnp.int32


def _build_cand_indices():
    ci = np.zeros(_NC, np.int32)
    cj = np.zeros(_NC, np.int32)
    valid = np.zeros(_NC, np.int32)
    t = 0
    for a in range(_K):
        nb = _K // (a + 1)
        if nb == 0:
            break
        ci[t:t + nb] = a
        cj[t:t + nb] = np.arange(nb, dtype=np.int32)
        valid[t:t + nb] = 1
        t += nb
    assert _K <= t <= _NC, t
    return ci, cj, valid


_CI_NP, _CJ_NP, _CVALID_NP = _build_cand_indices()


def _t128(row_chunk):
    return lax.dot_general(row_chunk, jnp.ones((1, 1), _F),
                           (((0,), (0,)), ((), ())),
                           precision=lax.Precision.HIGHEST,
                           preferred_element_type=_F)


def _topk_rank_scan(xr_ref, kr_get, ki_col_fn, n):
    nch = n // 128
    rk_iota = lax.broadcasted_iota(_I, (1, _K), 1)

    def c_step(c, accs):
        va, ka = accs
        xi = _t128(xr_ref[:, pl.ds(c * 128, 128)])
        ki = ki_col_fn(c)

        def d_step(d, acc):
            xj = xr_ref[:, pl.ds(d * 128, 128)]
            kj = kr_get(d)
            beats = (xj > xi) | ((xj == xi) & (kj < ki))
            return acc + beats.astype(_I)

        acc = lax.fori_loop(0, nch, d_step, jnp.zeros((128, 128), _I))
        rank = jnp.sum(acc, axis=1, keepdims=True)
        hit = rank == rk_iota
        va = va + jnp.sum(jnp.where(hit, xi, 0.0), axis=0, keepdims=True)
        ka = ka + jnp.sum(jnp.where(hit, ki, 0.0), axis=0, keepdims=True)
        return va, ka

    return lax.fori_loop(0, nch, c_step,
                         (jnp.zeros((1, _K), _F), jnp.zeros((1, _K), _F)))


def _topk_iota_body(xr_ref, vals_ref, idx_ref):
    def kr_get(d):
        return (lax.broadcasted_iota(_I, (1, 128), 1) + d * 128).astype(_F)

    def ki_col(c):
        return (lax.broadcasted_iota(_I, (128, 1), 0) + c * 128).astype(_F)

    va, ka = _topk_rank_scan(xr_ref, kr_get, ki_col, _V)
    vals_ref[...] = va
    idx_ref[...] = ka.astype(_I)


def _topk_keyed_body(xr_ref, kr_ref, vals_ref, key_ref):
    def kr_get(d):
        return kr_ref[:, pl.ds(d * 128, 128)]

    def ki_col(c):
        return _t128(kr_ref[:, pl.ds(c * 128, 128)])

    va, ka = _topk_rank_scan(xr_ref, kr_get, ki_col, _NC)
    vals_ref[...] = va
    key_ref[...] = ka.astype(_I)


def _cand_body(vr_ref, ur_ref, jr_ref, cir_ref, cjr_ref, valr_ref,
               cv_ref, ck_ref):
    ci = cir_ref[...]
    cj = cjr_ref[...]
    val = valr_ref[...]

    def step(kb, acc):
        av, au, aj = acc
        kc = lax.broadcasted_iota(_I, (128, 1), 0) + kb * 128
        vcb = _t128(vr_ref[:, pl.ds(kb * 128, 128)])
        ucb = _t128(ur_ref[:, pl.ds(kb * 128, 128)])
        jcb = _t128(jr_ref[:, pl.ds(kb * 128, 128)])
        mi = kc == ci
        mj = kc == cj
        av = av + jnp.sum(jnp.where(mi, vcb, 0.0), axis=0, keepdims=True)
        au = au + jnp.sum(jnp.where(mj, ucb, 0.0), axis=0, keepdims=True)
        aj = aj + jnp.sum(jnp.where(mj, jcb, 0.0), axis=0, keepdims=True)
        return av, au, aj

    av, au, aj = lax.fori_loop(
        0, _K // 128, step,
        (jnp.zeros((1, _NC), _F), jnp.zeros((1, _NC), _F),
         jnp.zeros((1, _NC), _F)))
    ok = val != 0
    cv_ref[...] = jnp.where(ok, av * au, -1.0)
    pad = (lax.broadcasted_iota(_I, (1, _NC), 1) + _V * _K).astype(_F)
    ck_ref[...] = jnp.where(ok, ci.astype(_F) * _V + aj, pad)


def _resolve_body(i1r_ref, sk2r_ref, sk3r_ref, s0_ref, s1_ref, s2_ref):
    sk3 = sk3r_ref[...]
    r3 = jnp.floor(sk3 * (1.0 / _V))
    c3 = sk3 - r3 * _V

    def step(kb, acc):
        e, s1 = acc
        kc = (lax.broadcasted_iota(_I, (128, 1), 0) + kb * 128).astype(_F)
        sk2c = _t128(sk2r_ref[:, pl.ds(kb * 128, 128)])
        a2c = jnp.floor(sk2c * (1.0 / _V))
        b2c = sk2c - a2c * _V
        m = kc == r3
        e = e + jnp.sum(jnp.where(m, a2c, 0.0), axis=0, keepdims=True)
        s1 = s1 + jnp.sum(jnp.where(m, b2c, 0.0), axis=0, keepdims=True)
        return e, s1

    e, s1 = lax.fori_loop(0, _K // 128, step,
                          (jnp.zeros((1, _K), _F), jnp.zeros((1, _K), _F)))

    def step2(kb, s0):
        kc = (lax.broadcasted_iota(_I, (128, 1), 0) + kb * 128).astype(_F)
        i1c = _t128(i1r_ref[:, pl.ds(kb * 128, 128)])
        m2 = kc == e
        return s0 + jnp.sum(jnp.where(m2, i1c, 0.0), axis=0, keepdims=True)

    s0 = lax.fori_loop(0, _K // 128, step2, jnp.zeros((1, _K), _F))
    s0_ref[...] = s0.astype(_I)
    s1_ref[...] = s1.astype(_I)
    s2_ref[...] = c3.astype(_I)


_RPB = 8


def _mat_body(v3_ref, s0_ref, s1_ref, s2_ref, p3_ref, probs_ref, syms_ref):
    p3v = p3_ref[...]
    G = _V // 128
    jpat = (lax.broadcasted_iota(_I, (G, 4, 128), 0) * 128
            + lax.broadcasted_iota(_I, (G, 4, 128), 2))
    cidx = lax.broadcasted_iota(_I, (G, 4, 128), 1)
    for rr in range(_RPB):
        v3 = v3_ref[0, 0, rr]
        probs_ref[pl.ds(rr * _V, _V)] = (v3 * p3v).reshape(_V)
        s0 = s0_ref[0, 0, rr]
        s1 = s1_ref[0, 0, rr]
        s2 = s2_ref[0, 0, rr]
        out = jnp.where(cidx == 0, s0,
                        jnp.where(cidx == 1, s1,
                                  jnp.where(cidx == 2, s2, jpat)))
        syms_ref[pl.ds(rr * G, G), :, :] = out


def kernel(p0, p1, p2, p3, k):
    del k

    topk_iota = pl.pallas_call(
        _topk_iota_body,
        out_shape=[jax.ShapeDtypeStruct((1, _K), _F),
                   jax.ShapeDtypeStruct((1, _K), _I)],
        interpret=_INTERP,
    )
    topk_keyed = pl.pallas_call(
        _topk_keyed_body,
        out_shape=[jax.ShapeDtypeStruct((1, _K), _F),
                   jax.ShapeDtypeStruct((1, _K), _I)],
        interpret=_INTERP,
    )
    cand = pl.pallas_call(
        _cand_body,
        out_shape=[jax.ShapeDtypeStruct((1, _NC), _F),
                   jax.ShapeDtypeStruct((1, _NC), _F)],
        interpret=_INTERP,
    )
    resolve = pl.pallas_call(
        _resolve_body,
        out_shape=[jax.ShapeDtypeStruct((1, _K), _I)] * 3,
        interpret=_INTERP,
    )
    mat = pl.pallas_call(
        _mat_body,
        grid=(_K // _RPB,),
        in_specs=[
            pl.BlockSpec((1, 1, _RPB), lambda i: (i, 0, 0)),
            pl.BlockSpec((1, 1, _RPB), lambda i: (i, 0, 0)),
            pl.BlockSpec((1, 1, _RPB), lambda i: (i, 0, 0)),
            pl.BlockSpec((1, 1, _RPB), lambda i: (i, 0, 0)),
            pl.BlockSpec((1, _V), lambda i: (0, 0)),
        ],
        out_specs=[
            pl.BlockSpec((_V * _RPB,), lambda i: (i,)),
            pl.BlockSpec((_V // 128 * _RPB, 4, 128), lambda i: (i, 0, 0)),
        ],
        out_shape=[jax.ShapeDtypeStruct((_K * _V,), _F),
                   jax.ShapeDtypeStruct((_K * _V // 128, 4, 128), _I)],
        interpret=_INTERP,
    )

    cir = jnp.asarray(_CI_NP).reshape(1, _NC)
    cjr = jnp.asarray(_CJ_NP).reshape(1, _NC)
    valr = jnp.asarray(_CVALID_NP).reshape(1, _NC)

    def row(a):
        return a.reshape(1, a.size)

    v1, i1 = topk_iota(row(p0))
    u1, j1 = topk_iota(row(p1))
    u2, j2 = topk_iota(row(p2))

    cv2, ck2 = cand(v1, u1, j1.astype(_F), cir, cjr, valr)
    v2, sk2 = topk_keyed(cv2, ck2)

    cv3, ck3 = cand(v2, u2, j2.astype(_F), cir, cjr, valr)
    v3, sk3 = topk_keyed(cv3, ck3)

    s0, s1, s2 = resolve(i1.astype(_F), sk2.astype(_F), sk3.astype(_F))

    def grp(a):
        return a.reshape(_K // _RPB, 1, _RPB)

    probs1d, syms3d = mat(grp(v3), grp(s0), grp(s1), grp(s2),
                          p3.reshape(1, _V))
    return probs1d, syms3d.transpose(0, 2, 1).reshape(_K * _V, 4)


# speedup vs baseline: 9.5350x; 2.6075x over previous
import numpy as np
import jax
import jax.numpy as jnp
from jax import lax
from jax.experimental import pallas as pl

_V = 8192
_K = 1024
_NC = 8192
_INTERP = False

_F = jnp.float32
_I = jnp.int32


def _build_cand_indices():
    ci = np.zeros(_NC, np.int32)
    cj = np.zeros(_NC, np.int32)
    valid = np.zeros(_NC, np.int32)
    t = 0
    for a in range(_K):
        nb = _K // (a + 1)
        if nb == 0:
            break
        ci[t:t + nb] = a
        cj[t:t + nb] = np.arange(nb, dtype=np.int32)
        valid[t:t + nb] = 1
        t += nb
    assert _K <= t <= _NC, t
    return ci, cj, valid


_CI_NP, _CJ_NP, _CVALID_NP = _build_cand_indices()


def _t128(row_chunk):
    return lax.dot_general(row_chunk, jnp.ones((1, 1), _F),
                           (((0,), (0,)), ((), ())),
                           precision=lax.Precision.HIGHEST,
                           preferred_element_type=_F)


def _topk_rank_scan(xr_get, kr_get, ki_col_fn, n):
    nch = n // 128
    rk_iota = lax.broadcasted_iota(_I, (1, _K), 1)

    def c_step(c, accs):
        va, ka = accs
        xi = _t128(xr_get(c))
        ki = ki_col_fn(c)

        def d_step(d, acc):
            xj = xr_get(d)
            kj = kr_get(d)
            beats = (xj > xi) | ((xj == xi) & (kj < ki))
            return acc + beats.astype(_I)

        acc = lax.fori_loop(0, nch, d_step, jnp.zeros((128, 128), _I),
                            unroll=8)
        rank = jnp.sum(acc, axis=1, keepdims=True)
        hit = rank == rk_iota
        va = va + jnp.sum(jnp.where(hit, xi, 0.0), axis=0, keepdims=True)
        ka = ka + jnp.sum(jnp.where(hit, ki, 0.0), axis=0, keepdims=True)
        return va, ka

    return lax.fori_loop(0, nch, c_step,
                         (jnp.zeros((1, _K), _F), jnp.zeros((1, _K), _F)))


def _topk_iota_body(xr_ref, vals_ref, idx_ref):
    def xr_get(d):
        return xr_ref[:, pl.ds(d * 128, 128)]

    def kr_get(d):
        return (lax.broadcasted_iota(_I, (1, 128), 1) + d * 128).astype(_F)

    def ki_col(c):
        return (lax.broadcasted_iota(_I, (128, 1), 0) + c * 128).astype(_F)

    va, ka = _topk_rank_scan(xr_get, kr_get, ki_col, _V)
    vals_ref[...] = va
    idx_ref[...] = ka.astype(_I)


def _topk_keyed_body(xr_ref, kr_ref, vals_ref, key_ref):
    def xr_get(d):
        return xr_ref[:, pl.ds(d * 128, 128)]

    def kr_get(d):
        return kr_ref[:, pl.ds(d * 128, 128)]

    def ki_col(c):
        return _t128(kr_ref[:, pl.ds(c * 128, 128)])

    va, ka = _topk_rank_scan(xr_get, kr_get, ki_col, _NC)
    vals_ref[...] = va
    key_ref[...] = ka.astype(_I)


def _cand_body(vr_ref, ur_ref, jr_ref, cir_ref, cjr_ref, valr_ref,
               cv_ref, ck_ref):
    ci = cir_ref[...]
    cj = cjr_ref[...]
    val = valr_ref[...]

    def step(kb, acc):
        av, au, aj = acc
        kc = lax.broadcasted_iota(_I, (128, 1), 0) + kb * 128
        vcb = _t128(vr_ref[:, pl.ds(kb * 128, 128)])
        ucb = _t128(ur_ref[:, pl.ds(kb * 128, 128)])
        jcb = _t128(jr_ref[:, pl.ds(kb * 128, 128)])
        mi = kc == ci
        mj = kc == cj
        av = av + jnp.sum(jnp.where(mi, vcb, 0.0), axis=0, keepdims=True)
        au = au + jnp.sum(jnp.where(mj, ucb, 0.0), axis=0, keepdims=True)
        aj = aj + jnp.sum(jnp.where(mj, jcb, 0.0), axis=0, keepdims=True)
        return av, au, aj

    av, au, aj = lax.fori_loop(
        0, _K // 128, step,
        (jnp.zeros((1, _NC), _F), jnp.zeros((1, _NC), _F),
         jnp.zeros((1, _NC), _F)))
    ok = val != 0
    cv_ref[...] = jnp.where(ok, av * au, -1.0)
    pad = (lax.broadcasted_iota(_I, (1, _NC), 1) + _V * _K).astype(_F)
    ck_ref[...] = jnp.where(ok, ci.astype(_F) * _V + aj, pad)


def _resolve_body(i1r_ref, sk2r_ref, sk3r_ref, s0_ref, s1_ref, s2_ref):
    sk3 = sk3r_ref[...]
    r3 = jnp.floor(sk3 * (1.0 / _V))
    c3 = sk3 - r3 * _V

    def step(kb, acc):
        e, s1 = acc
        kc = (lax.broadcasted_iota(_I, (128, 1), 0) + kb * 128).astype(_F)
        sk2c = _t128(sk2r_ref[:, pl.ds(kb * 128, 128)])
        a2c = jnp.floor(sk2c * (1.0 / _V))
        b2c = sk2c - a2c * _V
        m = kc == r3
        e = e + jnp.sum(jnp.where(m, a2c, 0.0), axis=0, keepdims=True)
        s1 = s1 + jnp.sum(jnp.where(m, b2c, 0.0), axis=0, keepdims=True)
        return e, s1

    e, s1 = lax.fori_loop(0, _K // 128, step,
                          (jnp.zeros((1, _K), _F), jnp.zeros((1, _K), _F)))

    def step2(kb, s0):
        kc = (lax.broadcasted_iota(_I, (128, 1), 0) + kb * 128).astype(_F)
        i1c = _t128(i1r_ref[:, pl.ds(kb * 128, 128)])
        m2 = kc == e
        return s0 + jnp.sum(jnp.where(m2, i1c, 0.0), axis=0, keepdims=True)

    s0 = lax.fori_loop(0, _K // 128, step2, jnp.zeros((1, _K), _F))
    s0_ref[...] = s0.astype(_I)
    s1_ref[...] = s1.astype(_I)
    s2_ref[...] = c3.astype(_I)


_RPB = 8


def _mat_body(v3_ref, s0_ref, s1_ref, s2_ref, p3_ref, probs_ref, syms_ref):
    p3v = p3_ref[...]
    G = _V // 128
    jpat = (lax.broadcasted_iota(_I, (G, 4, 128), 0) * 128
            + lax.broadcasted_iota(_I, (G, 4, 128), 2))
    cidx = lax.broadcasted_iota(_I, (G, 4, 128), 1)
    for rr in range(_RPB):
        v3 = v3_ref[0, 0, rr]
        probs_ref[pl.ds(rr * _V, _V)] = (v3 * p3v).reshape(_V)
        s0 = s0_ref[0, 0, rr]
        s1 = s1_ref[0, 0, rr]
        s2 = s2_ref[0, 0, rr]
        out = jnp.where(cidx == 0, s0,
                        jnp.where(cidx == 1, s1,
                                  jnp.where(cidx == 2, s2, jpat)))
        syms_ref[pl.ds(rr * G, G), :, :] = out


def kernel(p0, p1, p2, p3, k):
    del k

    topk_iota = pl.pallas_call(
        _topk_iota_body,
        out_shape=[jax.ShapeDtypeStruct((1, _K), _F),
                   jax.ShapeDtypeStruct((1, _K), _I)],
        interpret=_INTERP,
    )
    topk_keyed = pl.pallas_call(
        _topk_keyed_body,
        out_shape=[jax.ShapeDtypeStruct((1, _K), _F),
                   jax.ShapeDtypeStruct((1, _K), _I)],
        interpret=_INTERP,
    )
    cand = pl.pallas_call(
        _cand_body,
        out_shape=[jax.ShapeDtypeStruct((1, _NC), _F),
                   jax.ShapeDtypeStruct((1, _NC), _F)],
        interpret=_INTERP,
    )
    resolve = pl.pallas_call(
        _resolve_body,
        out_shape=[jax.ShapeDtypeStruct((1, _K), _I)] * 3,
        interpret=_INTERP,
    )
    mat = pl.pallas_call(
        _mat_body,
        grid=(_K // _RPB,),
        in_specs=[
            pl.BlockSpec((1, 1, _RPB), lambda i: (i, 0, 0)),
            pl.BlockSpec((1, 1, _RPB), lambda i: (i, 0, 0)),
            pl.BlockSpec((1, 1, _RPB), lambda i: (i, 0, 0)),
            pl.BlockSpec((1, 1, _RPB), lambda i: (i, 0, 0)),
            pl.BlockSpec((1, _V), lambda i: (0, 0)),
        ],
        out_specs=[
            pl.BlockSpec((_V * _RPB,), lambda i: (i,)),
            pl.BlockSpec((_V // 128 * _RPB, 4, 128), lambda i: (i, 0, 0)),
        ],
        out_shape=[jax.ShapeDtypeStruct((_K * _V,), _F),
                   jax.ShapeDtypeStruct((_K * _V // 128, 4, 128), _I)],
        interpret=_INTERP,
    )

    cir = jnp.asarray(_CI_NP).reshape(1, _NC)
    cjr = jnp.asarray(_CJ_NP).reshape(1, _NC)
    valr = jnp.asarray(_CVALID_NP).reshape(1, _NC)

    def row(a):
        return a.reshape(1, a.size)

    v1, i1 = topk_iota(row(p0))
    u1, j1 = topk_iota(row(p1))
    u2, j2 = topk_iota(row(p2))

    cv2, ck2 = cand(v1, u1, j1.astype(_F), cir, cjr, valr)
    v2, sk2 = topk_keyed(cv2, ck2)

    cv3, ck3 = cand(v2, u2, j2.astype(_F), cir, cjr, valr)
    v3, sk3 = topk_keyed(cv3, ck3)

    s0, s1, s2 = resolve(i1.astype(_F), sk2.astype(_F), sk3.astype(_F))

    def grp(a):
        return a.reshape(_K // _RPB, 1, _RPB)

    probs1d, syms3d = mat(grp(v3), grp(s0), grp(s1), grp(s2),
                          p3.reshape(1, _V))
    return probs1d, syms3d.transpose(0, 2, 1).reshape(_K * _V, 4)
